# Initial kernel scaffold; baseline (speedup 1.0000x reference)
import functools

import jax
import jax.numpy as jnp
from jax import lax
from jax.experimental import pallas as pl
from jax.experimental.pallas import tpu as pltpu
from jax.experimental.pallas import tpu_sc as plsc

NUM_OPS = 128
N = 16384 * 200
NC, NS, L = 2, 16, 16
NW = NC * NS
PER_W = N // NW
CH = 12800
NCH = PER_W // CH

_mesh = plsc.VectorSubcoreMesh(core_axis_name="c", subcore_axis_name="s")


@functools.partial(
    pl.kernel,
    out_type=jax.ShapeDtypeStruct((N,), jnp.int32),
    mesh=_mesh,
    scratch_types=[
        pltpu.VMEM((NUM_OPS,), jnp.int32),
        pltpu.VMEM((CH,), jnp.int32),
        pltpu.VMEM((CH,), jnp.int32),
    ],
)
def _lookup(op_hbm, table_hbm, out_hbm, table_v, idx_v, out_v):
    wid = lax.axis_index("s") * NC + lax.axis_index("c")
    pltpu.sync_copy(table_hbm, table_v)
    base0 = wid * PER_W
    for c in range(NCH):
        base = base0 + c * CH

        pltpu.sync_copy(op_hbm.at[pl.ds(base, CH)], idx_v)

        def body(i, carry):
            idx = idx_v[pl.ds(i * L, L)]
            out_v[pl.ds(i * L, L)] = plsc.load_gather(table_v, [idx])
            return carry

        lax.fori_loop(0, CH // L, body, 0)

        pltpu.sync_copy(out_v, out_hbm.at[pl.ds(base, CH)])


def kernel(operation, vocabulary_size):
    out = _lookup(operation.reshape(-1), vocabulary_size)
    return out.reshape(operation.shape)


# speedup vs baseline: 216.7658x; 216.7658x over previous
import functools

import jax
import jax.numpy as jnp
from jax import lax
from jax.experimental import pallas as pl
from jax.experimental.pallas import tpu as pltpu
from jax.experimental.pallas import tpu_sc as plsc

NUM_OPS = 128
N = 16384 * 200
NC, NS, L = 2, 16, 16
NW = NC * NS
PER_W = N // NW
CH = 12800
NCH = PER_W // CH

_mesh = plsc.VectorSubcoreMesh(core_axis_name="c", subcore_axis_name="s")


@functools.partial(
    pl.kernel,
    out_type=jax.ShapeDtypeStruct((N,), jnp.int32),
    mesh=_mesh,
    scratch_types=[
        pltpu.VMEM((NUM_OPS,), jnp.int32),
        pltpu.VMEM((CH,), jnp.int32),
        pltpu.VMEM((CH,), jnp.int32),
    ],
    compiler_params=pltpu.CompilerParams(needs_layout_passes=False),
)
def _lookup(op_hbm, table_hbm, out_hbm, table_v, idx_v, out_v):
    wid = lax.axis_index("s") * NC + lax.axis_index("c")
    pltpu.sync_copy(table_hbm, table_v)
    base0 = wid * PER_W
    for c in range(NCH):
        base = base0 + c * CH

        pltpu.sync_copy(op_hbm.at[pl.ds(base, CH)], idx_v)

        def body(i, carry):
            idx = idx_v[pl.ds(i * L, L)]
            out_v[pl.ds(i * L, L)] = plsc.load_gather(table_v, [idx])
            return carry

        lax.fori_loop(0, CH // L, body, 0)

        pltpu.sync_copy(out_v, out_hbm.at[pl.ds(base, CH)])


def kernel(operation, vocabulary_size):
    out = _lookup(operation.reshape(-1), vocabulary_size)
    return out.reshape(operation.shape)


# speedup vs baseline: 271.6472x; 1.2532x over previous
import functools

import jax
import jax.numpy as jnp
from jax import lax
from jax.experimental import pallas as pl
from jax.experimental.pallas import tpu as pltpu
from jax.experimental.pallas import tpu_sc as plsc

NUM_OPS = 128
N = 16384 * 200
NC, NS, L = 2, 16, 16
NW = NC * NS
PER_W = N // NW
CH = 12800
NCH = PER_W // CH

_mesh = plsc.VectorSubcoreMesh(core_axis_name="c", subcore_axis_name="s")


@functools.partial(
    pl.kernel,
    out_type=jax.ShapeDtypeStruct((N,), jnp.int32),
    mesh=_mesh,
    scratch_types=[
        pltpu.VMEM((NUM_OPS,), jnp.int32),
        pltpu.VMEM((CH,), jnp.int32),
        pltpu.VMEM((CH,), jnp.int32),
    ],
    compiler_params=pltpu.CompilerParams(needs_layout_passes=False),
)
def _lookup(op_hbm, table_hbm, out_hbm, table_v, idx_v, out_v):
    wid = lax.axis_index("s") * NC + lax.axis_index("c")
    pltpu.sync_copy(table_hbm, table_v)
    base0 = wid * PER_W
    for c in range(NCH):
        base = base0 + c * CH

        pltpu.sync_copy(op_hbm.at[pl.ds(base, CH)], idx_v)

        @plsc.parallel_loop(0, CH, step=L, unroll=8)
        def body(i):
            idx = idx_v[pl.ds(i, L)]
            out_v[pl.ds(i, L)] = plsc.load_gather(table_v, [idx])

        pltpu.sync_copy(out_v, out_hbm.at[pl.ds(base, CH)])


def kernel(operation, vocabulary_size):
    out = _lookup(operation.reshape(-1), vocabulary_size)
    return out.reshape(operation.shape)


# speedup vs baseline: 284.3087x; 1.0466x over previous
import functools

import jax
import jax.numpy as jnp
from jax import lax
from jax.experimental import pallas as pl
from jax.experimental.pallas import tpu as pltpu
from jax.experimental.pallas import tpu_sc as plsc

NUM_OPS = 128
N = 16384 * 200
NC, NS, L = 2, 16, 16
NW = NC * NS
PER_W = N // NW
CH = 12800
NCH = PER_W // CH

_mesh = plsc.VectorSubcoreMesh(core_axis_name="c", subcore_axis_name="s")


@functools.partial(
    pl.kernel,
    out_type=jax.ShapeDtypeStruct((N,), jnp.int32),
    mesh=_mesh,
    scratch_types=[
        pltpu.VMEM((NUM_OPS,), jnp.int32),
        pltpu.VMEM((2, CH), jnp.int32),
        pltpu.VMEM((2, CH), jnp.int32),
        pltpu.SemaphoreType.DMA,
        pltpu.SemaphoreType.DMA,
        pltpu.SemaphoreType.DMA,
        pltpu.SemaphoreType.DMA,
    ],
    compiler_params=pltpu.CompilerParams(needs_layout_passes=False),
)
def _lookup(op_hbm, table_hbm, out_hbm, table_v, idx_v, out_v,
            in_sem0, in_sem1, out_sem0, out_sem1):
    wid = lax.axis_index("s") * NC + lax.axis_index("c")
    pltpu.sync_copy(table_hbm, table_v)
    base0 = wid * PER_W
    in_sems = (in_sem0, in_sem1)
    out_sems = (out_sem0, out_sem1)

    def in_copy(c, buf):
        return pltpu.make_async_copy(
            op_hbm.at[pl.ds(base0 + c * CH, CH)], idx_v.at[buf], in_sems[buf])

    def out_copy(c, buf):
        return pltpu.make_async_copy(
            out_v.at[buf], out_hbm.at[pl.ds(base0 + c * CH, CH)], out_sems[buf])

    in_copy(0, 0).start()
    for c in range(NCH):
        buf = c & 1
        if c + 1 < NCH:
            in_copy(c + 1, 1 - buf).start()
        in_copy(c, buf).wait()
        if c >= 2:
            out_copy(c - 2, buf).wait()

        @plsc.parallel_loop(0, CH, step=L, unroll=8)
        def body(i):
            idx = idx_v[buf, pl.ds(i, L)]
            out_v[buf, pl.ds(i, L)] = plsc.load_gather(table_v, [idx])

        out_copy(c, buf).start()
    out_copy(NCH - 2, NCH & 1).wait()
    out_copy(NCH - 1, (NCH - 1) & 1).wait()


def kernel(operation, vocabulary_size):
    out = _lookup(operation.reshape(-1), vocabulary_size)
    return out.reshape(operation.shape)


# speedup vs baseline: 513.1822x; 1.8050x over previous
import functools

import jax
import jax.numpy as jnp
from jax import lax
from jax.experimental import pallas as pl
from jax.experimental.pallas import tpu as pltpu
from jax.experimental.pallas import tpu_sc as plsc

NUM_OPS = 128
ROWS, COLS = 16384, 200
NC, NS, L = 2, 16, 16
NW = NC * NS
R = ROWS // NW
RC = 64
NCH = R // RC
CSTARTS = tuple(range(0, COLS - L + 1, L)) + (COLS - L,)

_mesh = plsc.VectorSubcoreMesh(core_axis_name="c", subcore_axis_name="s")


@functools.partial(
    pl.kernel,
    out_type=jax.ShapeDtypeStruct((ROWS, COLS), jnp.int32),
    mesh=_mesh,
    scratch_types=[
        pltpu.VMEM((NUM_OPS,), jnp.int32),
        pltpu.VMEM((2, RC, COLS), jnp.int32),
        pltpu.VMEM((2, RC, COLS), jnp.int32),
        pltpu.SemaphoreType.DMA,
        pltpu.SemaphoreType.DMA,
        pltpu.SemaphoreType.DMA,
        pltpu.SemaphoreType.DMA,
    ],
    compiler_params=pltpu.CompilerParams(needs_layout_passes=False),
)
def _lookup(op_hbm, table_hbm, out_hbm, table_v, idx_v, out_v,
            in_sem0, in_sem1, out_sem0, out_sem1):
    wid = lax.axis_index("s") * NC + lax.axis_index("c")
    pltpu.sync_copy(table_hbm, table_v)
    row0 = wid * R
    in_sems = (in_sem0, in_sem1)
    out_sems = (out_sem0, out_sem1)

    def in_copy(ci, buf):
        return pltpu.make_async_copy(
            op_hbm.at[pl.ds(row0 + ci * RC, RC), :],
            idx_v.at[buf], in_sems[buf])

    def out_copy(ci, buf):
        return pltpu.make_async_copy(
            out_v.at[buf],
            out_hbm.at[pl.ds(row0 + ci * RC, RC), :], out_sems[buf])

    in_copy(0, 0).start()
    for ci in range(NCH):
        buf = ci & 1
        if ci + 1 < NCH:
            in_copy(ci + 1, 1 - buf).start()
        in_copy(ci, buf).wait()
        if ci >= 2:
            out_copy(ci - 2, buf).wait()

        @plsc.parallel_loop(0, RC, step=1, unroll=2)
        def body(r):
            for cs in CSTARTS:
                idx = idx_v[buf, r, pl.ds(cs, L)]
                out_v[buf, r, pl.ds(cs, L)] = plsc.load_gather(table_v, [idx])

        out_copy(ci, buf).start()
    out_copy(NCH - 2, NCH & 1).wait()
    out_copy(NCH - 1, (NCH - 1) & 1).wait()


def kernel(operation, vocabulary_size):
    return _lookup(operation, vocabulary_size)


# speedup vs baseline: 952.2360x; 1.8556x over previous
import functools

import jax
import jax.numpy as jnp
from jax import lax
from jax.experimental import pallas as pl
from jax.experimental.pallas import tpu as pltpu
from jax.experimental.pallas import tpu_sc as plsc

NUM_OPS = 128
ROWS, COLS = 200, 16384
NC, NS, L = 2, 16, 16
NW = NC * NS
CW = 128
PER_W = COLS // NW
NCH = PER_W // CW

_mesh = plsc.VectorSubcoreMesh(core_axis_name="c", subcore_axis_name="s")


@functools.partial(
    pl.kernel,
    out_type=jax.ShapeDtypeStruct((ROWS, COLS), jnp.int32),
    mesh=_mesh,
    scratch_types=[
        pltpu.VMEM((NUM_OPS,), jnp.int32),
        pltpu.VMEM((2, ROWS, CW), jnp.int32),
        pltpu.VMEM((2, ROWS, CW), jnp.int32),
        pltpu.SemaphoreType.DMA,
        pltpu.SemaphoreType.DMA,
        pltpu.SemaphoreType.DMA,
        pltpu.SemaphoreType.DMA,
    ],
    compiler_params=pltpu.CompilerParams(needs_layout_passes=False),
)
def _lookup(op_hbm, table_hbm, out_hbm, table_v, idx_v, out_v,
            in_sem0, in_sem1, out_sem0, out_sem1):
    wid = lax.axis_index("s") * NC + lax.axis_index("c")
    pltpu.sync_copy(table_hbm, table_v)
    col0 = wid * PER_W
    in_sems = (in_sem0, in_sem1)
    out_sems = (out_sem0, out_sem1)

    def in_copy(ci, buf):
        return pltpu.make_async_copy(
            op_hbm.at[:, pl.ds(col0 + ci * CW, CW)],
            idx_v.at[buf], in_sems[buf])

    def out_copy(ci, buf):
        return pltpu.make_async_copy(
            out_v.at[buf],
            out_hbm.at[:, pl.ds(col0 + ci * CW, CW)], out_sems[buf])

    in_copy(0, 0).start()
    for ci in range(NCH):
        buf = ci & 1
        if ci + 1 < NCH:
            in_copy(ci + 1, 1 - buf).start()
        in_copy(ci, buf).wait()
        if ci >= 2:
            out_copy(ci - 2, buf).wait()

        @plsc.parallel_loop(0, ROWS, step=1, unroll=2)
        def body(r):
            for k in range(CW // L):
                idx = idx_v[buf, r, pl.ds(k * L, L)]
                out_v[buf, r, pl.ds(k * L, L)] = plsc.load_gather(table_v, [idx])

        out_copy(ci, buf).start()
    out_copy(NCH - 2, NCH & 1).wait()
    out_copy(NCH - 1, (NCH - 1) & 1).wait()


def kernel(operation, vocabulary_size):
    out_t = _lookup(operation.T, vocabulary_size)
    return out_t.T
